# initial kernel scaffold (unmeasured)
import jax
import jax.numpy as jnp
from jax import lax
from jax.experimental import pallas as pl
from jax.experimental.pallas import tpu as pltpu

N_DEV = 4
M_PER = 1024
K = 4096
K_PER = 1024
N_OUT = 8192
KT = 512
SUBS = K_PER // KT


def kernel(x, w_mat, scale_x, scale_w):
    fp8 = jnp.float8_e4m3fn

    def body(x_ref, w_hbm, sx_ref, sw_ref, out_ref,
             xs_ref, xr_ref, wv_ref, send_sems, recv_sems, w_sem):
        me = lax.axis_index("i")

        bsem = pltpu.get_barrier_semaphore()
        for off in range(1, N_DEV):
            peer = lax.rem(me + off, N_DEV)
            pl.semaphore_signal(bsem, inc=1, device_id=(peer,),
                                device_id_type=pl.DeviceIdType.MESH)
        pl.semaphore_wait(bsem, N_DEV - 1)

        rdmas = []
        for off in range(1, N_DEV):
            d = lax.rem(me + off, N_DEV)
            xs_ref[off] = x_ref[pl.ds(d * M_PER, M_PER), :].astype(fp8)
            rdma = pltpu.make_async_remote_copy(
                src_ref=xs_ref.at[off],
                dst_ref=xr_ref.at[off],
                send_sem=send_sems.at[off - 1],
                recv_sem=recv_sems.at[off - 1],
                device_id=(d,),
                device_id_type=pl.DeviceIdType.MESH,
            )
            rdma.start()
            rdmas.append(rdma)

        xr_ref[0] = x_ref[pl.ds(me * M_PER, M_PER), :].astype(fp8)

        first = True
        for t in (0, 1, 3, 2):
            if t > 0:
                rdmas[t - 1].wait_recv()
            j = lax.rem(me - t + N_DEV, N_DEV)
            for sub in range(SUBS):
                row = j * K_PER + sub * KT
                cp = pltpu.make_async_copy(
                    w_hbm.at[pl.ds(row, KT), :], wv_ref, w_sem)
                cp.start()
                cp.wait()
                a = xr_ref[t, :, pl.ds(sub * KT, KT)].astype(jnp.bfloat16)
                b = wv_ref[...].astype(jnp.bfloat16)
                part = lax.dot_general(
                    a, b, (((1,), (0,)), ((), ())),
                    preferred_element_type=jnp.float32)
                if first:
                    out_ref[...] = part
                    first = False
                else:
                    out_ref[...] += part

        for rdma in rdmas:
            rdma.wait_send()

        s = sx_ref[0] * sw_ref[0]
        y = out_ref[...] * s
        out_ref[...] = y / (1.0 + jnp.exp(-jnp.clip(y, -60.0, 60.0)))

    return pl.pallas_call(
        body,
        out_shape=jax.ShapeDtypeStruct((M_PER, N_OUT), jnp.float32),
        in_specs=[
            pl.BlockSpec(memory_space=pltpu.VMEM),
            pl.BlockSpec(memory_space=pltpu.ANY),
            pl.BlockSpec(memory_space=pltpu.SMEM),
            pl.BlockSpec(memory_space=pltpu.SMEM),
        ],
        out_specs=pl.BlockSpec(memory_space=pltpu.VMEM),
        scratch_shapes=[
            pltpu.VMEM((N_DEV, M_PER, K_PER), fp8),
            pltpu.VMEM((N_DEV, M_PER, K_PER), fp8),
            pltpu.VMEM((KT, N_OUT), jnp.float32),
            pltpu.SemaphoreType.DMA((N_DEV - 1,)),
            pltpu.SemaphoreType.DMA((N_DEV - 1,)),
            pltpu.SemaphoreType.DMA,
        ],
        compiler_params=pltpu.CompilerParams(collective_id=0),
    )(x, w_mat, scale_x, scale_w)


# baseline (device time: 185919 ns/iter reference)
import jax
import jax.numpy as jnp
from jax import lax
from jax.experimental import pallas as pl
from jax.experimental.pallas import tpu as pltpu

N_DEV = 4
M_PER = 1024
K = 4096
K_PER = 1024
N_OUT = 8192
KT = 256
SUBS = K_PER // KT


def kernel(x, w_mat, scale_x, scale_w):
    fp8 = jnp.float8_e4m3fn

    def body(x_hbm, w_hbm, sx_ref, sw_ref, out_ref,
             xs_ref, xr_ref, xstage_ref, wv_ref,
             send_sems, recv_sems, x_sem, w_sem):
        me = lax.axis_index("i")

        bsem = pltpu.get_barrier_semaphore()
        for off in range(1, N_DEV):
            peer = lax.rem(me + off, N_DEV)
            pl.semaphore_signal(bsem, inc=1, device_id=(peer,),
                                device_id_type=pl.DeviceIdType.MESH)
        pl.semaphore_wait(bsem, N_DEV - 1)

        rdmas = []
        for off in range(1, N_DEV):
            d = lax.rem(me + off, N_DEV)
            cp = pltpu.make_async_copy(
                x_hbm.at[pl.ds(d * M_PER, M_PER), :], xstage_ref, x_sem)
            cp.start()
            cp.wait()
            xs_ref[off] = xstage_ref[...].astype(fp8)
            rdma = pltpu.make_async_remote_copy(
                src_ref=xs_ref.at[off],
                dst_ref=xr_ref.at[off],
                send_sem=send_sems.at[off - 1],
                recv_sem=recv_sems.at[off - 1],
                device_id=(d,),
                device_id_type=pl.DeviceIdType.MESH,
            )
            rdma.start()
            rdmas.append(rdma)

        cp = pltpu.make_async_copy(
            x_hbm.at[pl.ds(me * M_PER, M_PER), :], xstage_ref, x_sem)
        cp.start()
        cp.wait()
        xr_ref[0] = xstage_ref[...].astype(fp8)

        first = True
        for t in (0, 1, 3, 2):
            if t > 0:
                rdmas[t - 1].wait_recv()
            j = lax.rem(me - t + N_DEV, N_DEV)
            for sub in range(SUBS):
                row = j * K_PER + sub * KT
                cp = pltpu.make_async_copy(
                    w_hbm.at[pl.ds(row, KT), :], wv_ref, w_sem)
                cp.start()
                cp.wait()
                a = xr_ref[t, :, pl.ds(sub * KT, KT)].astype(jnp.bfloat16)
                b = wv_ref[...].astype(jnp.bfloat16)
                part = lax.dot_general(
                    a, b, (((1,), (0,)), ((), ())),
                    preferred_element_type=jnp.float32)
                if first:
                    out_ref[...] = part
                    first = False
                else:
                    out_ref[...] += part

        for rdma in rdmas:
            rdma.wait_send()

        s = sx_ref[0] * sw_ref[0]
        for nc in range(8):
            y = out_ref[:, pl.ds(nc * (N_OUT // 8), N_OUT // 8)] * s
            out_ref[:, pl.ds(nc * (N_OUT // 8), N_OUT // 8)] = (
                y / (1.0 + jnp.exp(-jnp.clip(y, -60.0, 60.0))))

    return pl.pallas_call(
        body,
        out_shape=jax.ShapeDtypeStruct((M_PER, N_OUT), jnp.float32),
        in_specs=[
            pl.BlockSpec(memory_space=pltpu.MemorySpace.HBM),
            pl.BlockSpec(memory_space=pltpu.MemorySpace.HBM),
            pl.BlockSpec(memory_space=pltpu.SMEM),
            pl.BlockSpec(memory_space=pltpu.SMEM),
        ],
        out_specs=pl.BlockSpec(memory_space=pltpu.VMEM),
        scratch_shapes=[
            pltpu.VMEM((N_DEV, M_PER, K_PER), fp8),
            pltpu.VMEM((N_DEV, M_PER, K_PER), fp8),
            pltpu.VMEM((M_PER, K_PER), jnp.float32),
            pltpu.VMEM((KT, N_OUT), jnp.float32),
            pltpu.SemaphoreType.DMA((N_DEV - 1,)),
            pltpu.SemaphoreType.DMA((N_DEV - 1,)),
            pltpu.SemaphoreType.DMA,
            pltpu.SemaphoreType.DMA,
        ],
        compiler_params=pltpu.CompilerParams(
            collective_id=0, vmem_limit_bytes=60 * 1024 * 1024),
    )(x, w_mat, scale_x, scale_w)


def _warm_compile_cache():
    import json
    import os
    from pathlib import Path

    if os.environ.get("GENDIST_SKIP_WARM") == "1":
        return
    try:
        jax.config.update("jax_compilation_cache_dir", "/tmp/jax_cache")
        jax.config.update("jax_persistent_cache_min_compile_time_secs", 0.0)
        jax.config.update("jax_persistent_cache_min_entry_size_bytes", 0)

        import distributed_mesh_v7x as dm
        from jax.experimental.shard_map import shard_map
        from jax.sharding import NamedSharding

        here = Path(__file__).parent
        meta = json.loads((here / "mesh_meta.json").read_text())
        mesh = dm.get_mesh(meta["mesh_spec"], world_size=meta["world_size"])
        specs = meta["sharding_specs"]
        in_names = [k for k in specs if k != "__output__"]
        in_p = {k: dm.spec_from_json(specs[k]) for k in in_names}
        out_p = dm.spec_from_json(specs["__output__"])

        shapes = {"x": (4096, 4096), "w_mat": (4096, 8192),
                  "scale_x": (1,), "scale_w": (1,)}
        sds = tuple(
            jax.ShapeDtypeStruct(shapes[k], jnp.float32,
                                 sharding=NamedSharding(mesh, in_p[k]))
            for k in in_names)
        wrapped = jax.jit(shard_map(
            kernel, mesh=mesh,
            in_specs=tuple(in_p[k] for k in in_names),
            out_specs=out_p, check_rep=False))
        wrapped.lower(*sds).compile()
    except Exception as e:
        import sys
        print(f"[kernel warmup] skipped/failed: {type(e).__name__}: {e}",
              file=sys.stderr)


_warm_compile_cache()


# device time: 135111 ns/iter; 1.3760x vs baseline; 1.3760x over previous
import jax
import jax.numpy as jnp
from jax import lax
from jax.experimental import pallas as pl
from jax.experimental.pallas import tpu as pltpu

N_DEV = 4
M_PER = 1024
K = 4096
K_PER = 1024
N_OUT = 8192
KT = 128
SUBS = K_PER // KT
ORDER = (0, 1, 3, 2)


def kernel(x, w_mat, scale_x, scale_w):
    fp8 = jnp.float8_e4m3fn

    def body(x_hbm, w_hbm, sx_ref, sw_ref, out_ref,
             xs_ref, xr_ref, xstage_ref, wv_ref, b8_ref,
             send_sems, recv_sems, x_sem, w_sems):
        me = lax.axis_index("i")

        bsem = pltpu.get_barrier_semaphore()
        for off in range(1, N_DEV):
            peer = lax.rem(me + off, N_DEV)
            pl.semaphore_signal(bsem, inc=1, device_id=(peer,),
                                device_id_type=pl.DeviceIdType.MESH)
        pl.semaphore_wait(bsem, N_DEV - 1)

        def start_w(ti):
            t = ORDER[ti // SUBS]
            j = lax.rem(me - t + N_DEV, N_DEV)
            row = j * K_PER + (ti % SUBS) * KT
            buf = ti % 2
            cp = pltpu.make_async_copy(
                w_hbm.at[pl.ds(row, KT), :], wv_ref.at[buf],
                w_sems.at[buf])
            cp.start()
            return cp

        cps = {0: start_w(0), 1: start_w(1)}

        rdmas = [None] * N_DEV
        for off in (1, 3, 2):
            d = lax.rem(me + off, N_DEV)
            cp = pltpu.make_async_copy(
                x_hbm.at[pl.ds(d * M_PER, M_PER), :], xstage_ref, x_sem)
            cp.start()
            cp.wait()
            xs_ref[off - 1] = xstage_ref[...].astype(fp8)
            rdma = pltpu.make_async_remote_copy(
                src_ref=xs_ref.at[off - 1],
                dst_ref=xr_ref.at[off],
                send_sem=send_sems.at[off - 1],
                recv_sem=recv_sems.at[off - 1],
                device_id=(d,),
                device_id_type=pl.DeviceIdType.MESH,
            )
            rdma.start()
            rdmas[off] = rdma

        cp = pltpu.make_async_copy(
            x_hbm.at[pl.ds(me * M_PER, M_PER), :], xstage_ref, x_sem)
        cp.start()
        cp.wait()
        xr_ref[0] = xstage_ref[...].astype(fp8)

        for pos, t in enumerate(ORDER):
            if t > 0:
                rdmas[t].wait_recv()
            for sub in range(SUBS):
                ti = pos * SUBS + sub
                cps[ti].wait()
                b8_ref[pl.ds(sub * KT, KT), :] = (
                    wv_ref[ti % 2].astype(fp8))
                nxt = ti + 2
                if nxt < N_DEV * SUBS:
                    cps[nxt] = start_w(nxt)
            part = lax.dot_general(
                xr_ref[t], b8_ref[...], (((1,), (0,)), ((), ())),
                preferred_element_type=jnp.float32)
            if pos == 0:
                out_ref[...] = part
            else:
                out_ref[...] += part

        for off in (1, 2, 3):
            rdmas[off].wait_send()

        s = sx_ref[0] * sw_ref[0]
        for nc in range(8):
            y = out_ref[:, pl.ds(nc * (N_OUT // 8), N_OUT // 8)] * s
            out_ref[:, pl.ds(nc * (N_OUT // 8), N_OUT // 8)] = (
                y / (1.0 + jnp.exp(-jnp.clip(y, -60.0, 60.0))))

    return pl.pallas_call(
        body,
        out_shape=jax.ShapeDtypeStruct((M_PER, N_OUT), jnp.float32),
        in_specs=[
            pl.BlockSpec(memory_space=pltpu.MemorySpace.HBM),
            pl.BlockSpec(memory_space=pltpu.MemorySpace.HBM),
            pl.BlockSpec(memory_space=pltpu.SMEM),
            pl.BlockSpec(memory_space=pltpu.SMEM),
        ],
        out_specs=pl.BlockSpec(memory_space=pltpu.VMEM),
        scratch_shapes=[
            pltpu.VMEM((N_DEV - 1, M_PER, K_PER), fp8),
            pltpu.VMEM((N_DEV, M_PER, K_PER), fp8),
            pltpu.VMEM((M_PER, K_PER), jnp.float32),
            pltpu.VMEM((2, KT, N_OUT), jnp.float32),
            pltpu.VMEM((K_PER, N_OUT), fp8),
            pltpu.SemaphoreType.DMA((N_DEV - 1,)),
            pltpu.SemaphoreType.DMA((N_DEV - 1,)),
            pltpu.SemaphoreType.DMA,
            pltpu.SemaphoreType.DMA((2,)),
        ],
        compiler_params=pltpu.CompilerParams(
            collective_id=0, vmem_limit_bytes=62 * 1024 * 1024),
    )(x, w_mat, scale_x, scale_w)


def _warm_compile_cache():
    import json
    import os
    from pathlib import Path

    if os.environ.get("GENDIST_SKIP_WARM") == "1":
        return
    try:
        jax.config.update("jax_compilation_cache_dir", "/tmp/jax_cache")
        jax.config.update("jax_persistent_cache_min_compile_time_secs", 0.0)
        jax.config.update("jax_persistent_cache_min_entry_size_bytes", 0)

        import distributed_mesh_v7x as dm
        from jax.experimental.shard_map import shard_map
        from jax.sharding import NamedSharding

        here = Path(__file__).parent
        meta = json.loads((here / "mesh_meta.json").read_text())
        mesh = dm.get_mesh(meta["mesh_spec"], world_size=meta["world_size"])
        specs = meta["sharding_specs"]
        in_names = [k for k in specs if k != "__output__"]
        in_p = {k: dm.spec_from_json(specs[k]) for k in in_names}
        out_p = dm.spec_from_json(specs["__output__"])

        shapes = {"x": (4096, 4096), "w_mat": (4096, 8192),
                  "scale_x": (1,), "scale_w": (1,)}
        sds = tuple(
            jax.ShapeDtypeStruct(shapes[k], jnp.float32,
                                 sharding=NamedSharding(mesh, in_p[k]))
            for k in in_names)
        wrapped = jax.jit(shard_map(
            kernel, mesh=mesh,
            in_specs=tuple(in_p[k] for k in in_names),
            out_specs=out_p, check_rep=False))
        wrapped.lower(*sds).compile()
    except Exception as e:
        import sys
        print(f"[kernel warmup] skipped/failed: {type(e).__name__}: {e}",
              file=sys.stderr)


_warm_compile_cache()
